# baseline (speedup 1.0000x reference)
import functools

import jax
import jax.numpy as jnp
from jax import lax
from jax.experimental import pallas as pl
from jax.experimental.pallas import tpu as pltpu
from jax.experimental.pallas import tpu_sc as plsc

NUM_GLYPHS = 5976
ENT_PAD = 2048
NGRP = 13
D = 64
NC, NS = 2, 16
NW = NC * NS
CH = 128
N_TOTAL = 1024 * 200
NPW = N_TOTAL // NW
NCH = NPW // CH


def _make_lookup():
    mesh = plsc.VectorSubcoreMesh(
        core_axis_name="c", subcore_axis_name="s",
        num_cores=NC, num_subcores=NS)

    @functools.partial(
        pl.kernel, mesh=mesh,
        compiler_params=pltpu.CompilerParams(
            needs_layout_passes=False, use_tc_tiling_on_sc=False),
        out_type=(
            jax.ShapeDtypeStruct((N_TOTAL // CH, CH, D), jnp.float32),
            jax.ShapeDtypeStruct((NC * NGRP * ENT_PAD, D), jnp.float32),
        ),
        scratch_types=[
            pltpu.VMEM((NPW,), jnp.int32),
            pltpu.VMEM((NUM_GLYPHS,), jnp.int32),
            pltpu.VMEM((NUM_GLYPHS,), jnp.int32),
            pltpu.VMEM((CH, D), jnp.float32),
            pltpu.VMEM((NGRP, D), jnp.float32),
            pltpu.VMEM((2, CH, D), jnp.float32),
            pltpu.VMEM((4, CH), jnp.int32),
            pltpu.VMEM((4, CH, D), jnp.float32),
            pltpu.SemaphoreType.DMA,
            pltpu.SemaphoreType.DMA,
            pltpu.SemaphoreType.DMA,
            pltpu.SemaphoreType.DMA,
            pltpu.SemaphoreType.DMA,
            pltpu.SemaphoreType.DMA,
            pltpu.SemaphoreType.DMA,
            pltpu.SemaphoreType.DMA,
            pltpu.SemaphoreType.DMA,
            pltpu.SemaphoreType.DMA,
        ],
    )
    def lookup(ent_hbm, grp_hbm, elut_hbm, glut_hbm, gl_hbm,
               out_hbm, ct_hbm,
               gl_v, elut_v, glut_v, ent_v, grp_v, bld_v, idx_v, rows_v,
               bsem0, bsem1,
               gsem0, gsem1, gsem2, gsem3, wsem0, wsem1, wsem2, wsem3):
        cid = lax.axis_index("c")
        sid = lax.axis_index("s")
        wid = sid * NC + cid
        cbase = cid * (NGRP * ENT_PAD)

        erow = pl.multiple_of(sid * CH, CH)
        pltpu.sync_copy(ent_hbm.at[pl.ds(erow, CH)], ent_v)
        pltpu.sync_copy(grp_hbm, grp_v)
        bsems = (bsem0, bsem1)

        def bld_dst(g, slot):
            row = cbase + g * ENT_PAD + erow
            return (bld_v.at[slot], ct_hbm.at[pl.ds(row, CH)])

        for g in range(NGRP):
            slot = g % 2
            if g >= 2:
                src, dst = bld_dst(g - 2, slot)
                pltpu.make_async_copy(src, dst, bsems[slot]).wait()

            def row_block(rb, carry, g=g, slot=slot):
                r0 = pl.multiple_of(rb * 8, 8)
                for r in range(8):
                    for q in range(D // 16):
                        cc = q * 16
                        bld_v[slot, r0 + r, pl.ds(cc, 16)] = (
                            ent_v[r0 + r, pl.ds(cc, 16)]
                            + grp_v[g, pl.ds(cc, 16)])
                return carry

            lax.fori_loop(0, CH // 8, row_block, 0)
            src, dst = bld_dst(g, slot)
            pltpu.async_copy(src, dst, bsems[slot])
        for g in (NGRP - 2, NGRP - 1):
            src, dst = bld_dst(g, g % 2)
            pltpu.make_async_copy(src, dst, bsems[g % 2]).wait()
        plsc.subcore_barrier()

        base = pl.multiple_of(wid * NPW, NPW)
        kbase = pl.multiple_of(wid * NCH, NCH)
        pltpu.sync_copy(gl_hbm.at[pl.ds(base, NPW)], gl_v)
        pltpu.sync_copy(elut_hbm, elut_v)
        pltpu.sync_copy(glut_hbm, glut_v)
        gsems = (gsem0, gsem1, gsem2, gsem3)
        wsems = (wsem0, wsem1, wsem2, wsem3)

        def indices(j, slot):
            off = pl.multiple_of(j * CH, CH)
            for t in range(CH // 16):
                g = gl_v[pl.ds(off + t * 16, 16)]
                ge = plsc.load_gather(elut_v, [g])
                gg = plsc.load_gather(glut_v, [g])
                idx_v[slot, pl.ds(t * 16, 16)] = cbase + gg * ENT_PAD + ge

        def gather_start(slot):
            pltpu.async_copy(ct_hbm.at[idx_v.at[slot]], rows_v.at[slot],
                             gsems[slot])

        def gather_wait(slot):
            pltpu.make_async_copy(ct_hbm.at[idx_v.at[slot]],
                                  rows_v.at[slot], gsems[slot]).wait()

        def write_start(slot, j):
            pltpu.async_copy(rows_v.at[slot], out_hbm.at[kbase + j],
                             wsems[slot])

        def write_wait(slot, j):
            pltpu.make_async_copy(rows_v.at[slot], out_hbm.at[kbase + j],
                                  wsems[slot]).wait()

        for j in range(2):
            indices(j, j % 4)
            gather_start(j % 4)
        for j in range(NCH):
            jn = j + 2
            if jn < NCH:
                sn = jn % 4
                indices(jn, sn)
                if jn - 4 >= 0:
                    write_wait(sn, jn - 4)
                gather_start(sn)
            gather_wait(j % 4)
            write_start(j % 4, j)
        for j in range(NCH - 4, NCH):
            write_wait(j % 4, j)

    return lookup


_lookup = _make_lookup()


def kernel(glyphs, entity_lut, group_lut, entity_table, group_table):
    b, s = glyphs.shape
    gl = glyphs.astype(jnp.int32).reshape(b * s)
    elut = entity_lut.astype(jnp.int32)
    glut = group_lut.astype(jnp.int32)
    ent_p = jnp.pad(entity_table,
                    ((0, ENT_PAD - entity_table.shape[0]), (0, 0)))
    out, _ = _lookup(ent_p, group_table, elut, glut, gl)
    return out.reshape(b, s, D)
